# Initial kernel scaffold; baseline (speedup 1.0000x reference)
import jax
import jax.numpy as jnp
from jax.experimental import pallas as pl


def kernel(x, edge_index, W, b):
    raise NotImplementedError("write your pallas kernel here")


# speedup vs baseline: 27.2681x; 27.2681x over previous
import functools

import jax
import jax.numpy as jnp
from jax import lax
from jax.experimental import pallas as pl
from jax.experimental.pallas import tpu as pltpu
from jax.experimental.pallas import tpu_sc as plsc

NC = 2
NS = 16
NW = NC * NS


def _round_up(a, m):
    return -(-a // m) * m


def _plan_edges(e):
    for chunk in range(128, 0, -8):
        if e % (NW * chunk) == 0:
            return chunk, e // (NW * chunk), 0
    chunk = 128
    nch = -(-e // (NW * chunk))
    return chunk, nch, NW * chunk * nch - e


def _sc_degree(dst3, nrows):
    nw, nch, chunk = dst3.shape
    rpt = nrows // NS

    @functools.partial(
        pl.kernel,
        out_type=jax.ShapeDtypeStruct((NC, nrows), jnp.float32),
        mesh=plsc.VectorSubcoreMesh(core_axis_name="c", subcore_axis_name="s"),
        scratch_types=[
            pltpu.VMEM((nch, chunk), jnp.int32),
            pltpu.VMEM((chunk,), jnp.float32),
            pltpu.VMEM((rpt,), jnp.float32),
            pltpu.VMEM_SHARED((nrows,), jnp.float32),
        ],
    )
    def deg_kernel(dst_hbm, degp_hbm, idx_v, ones_v, zb_v, deg_sh):
        c = lax.axis_index("c")
        s = lax.axis_index("s")
        w = c * NS + s
        pltpu.sync_copy(dst_hbm.at[w], idx_v)

        @pl.loop(0, chunk // 16)
        def _ones(i):
            ones_v[pl.ds(i * 16, 16)] = jnp.ones((16,), jnp.float32)

        @pl.loop(0, rpt // 16)
        def _zb(i):
            zb_v[pl.ds(i * 16, 16)] = jnp.zeros((16,), jnp.float32)

        pltpu.sync_copy(zb_v, deg_sh.at[pl.ds(s * rpt, rpt)])
        plsc.subcore_barrier()

        @pl.loop(0, nch)
        def _scat(j):
            pltpu.sync_copy(ones_v, deg_sh.at[idx_v.at[j]], add=True)

        plsc.subcore_barrier()
        pltpu.sync_copy(deg_sh.at[pl.ds(s * rpt, rpt)], zb_v)
        pltpu.sync_copy(zb_v, degp_hbm.at[c, pl.ds(s * rpt, rpt)])

    return deg_kernel(dst3)


def _sc_scatter(g, src3, dst3, nrows):
    nw, nch, chunk = src3.shape
    d = g.shape[1]
    rpt = nrows // NS
    nfull = rpt // chunk
    rem = rpt % chunk

    @functools.partial(
        pl.kernel,
        out_type=jax.ShapeDtypeStruct((NC, nrows, d), jnp.float32),
        mesh=plsc.VectorSubcoreMesh(core_axis_name="c", subcore_axis_name="s"),
        scratch_types=[
            pltpu.VMEM((nch, chunk), jnp.int32),
            pltpu.VMEM((nch, chunk), jnp.int32),
            pltpu.VMEM((chunk, d), jnp.float32),
            pltpu.VMEM_SHARED((nrows, d), jnp.float32),
            pltpu.SemaphoreType.DMA,
        ],
    )
    def scat_kernel(g_hbm, src_hbm, dst_hbm, acc_hbm,
                    src_v, dst_v, buf_v, acc_sh, sem):
        c = lax.axis_index("c")
        s = lax.axis_index("s")
        w = c * NS + s
        pltpu.sync_copy(src_hbm.at[w], src_v)
        pltpu.sync_copy(dst_hbm.at[w], dst_v)

        @pl.loop(0, chunk)
        def _zr(r):
            @pl.loop(0, d // 16)
            def _zc(i):
                buf_v[r, pl.ds(i * 16, 16)] = jnp.zeros((16,), jnp.float32)

        base = s * rpt

        @pl.loop(0, nfull)
        def _za(k):
            pltpu.sync_copy(buf_v, acc_sh.at[pl.ds(base + k * chunk, chunk)])

        if rem:
            pltpu.sync_copy(buf_v.at[pl.ds(0, rem)],
                            acc_sh.at[pl.ds(base + nfull * chunk, rem)])

        plsc.subcore_barrier()

        @pl.loop(0, nch)
        def _edge(j):
            pltpu.async_copy(g_hbm.at[src_v.at[j]], buf_v, sem).wait()
            pltpu.sync_copy(buf_v, acc_sh.at[dst_v.at[j]], add=True)

        plsc.subcore_barrier()

        @pl.loop(0, nfull)
        def _out(k):
            pltpu.sync_copy(acc_sh.at[pl.ds(base + k * chunk, chunk)], buf_v)
            pltpu.sync_copy(buf_v, acc_hbm.at[c, pl.ds(base + k * chunk, chunk)])

        if rem:
            pltpu.sync_copy(acc_sh.at[pl.ds(base + nfull * chunk, rem)],
                            buf_v.at[pl.ds(0, rem)])
            pltpu.sync_copy(buf_v.at[pl.ds(0, rem)],
                            acc_hbm.at[c, pl.ds(base + nfull * chunk, rem)])

    return scat_kernel(g, src3, dst3)


def _tca_body(x_ref, w_ref, degp_ref, g_ref):
    deg = jnp.sum(degp_ref[...], axis=0) + 1.0
    dinv = lax.rsqrt(deg)
    h = jnp.dot(x_ref[...], w_ref[...], preferred_element_type=jnp.float32)
    g_ref[...] = h * dinv[:, None]


def _tc_transform(x, W, degp):
    n, d_in = x.shape
    d = W.shape[1]
    br = 512
    return pl.pallas_call(
        _tca_body,
        grid=(pl.cdiv(n, br),),
        in_specs=[
            pl.BlockSpec((br, d_in), lambda i: (i, 0)),
            pl.BlockSpec((d_in, d), lambda i: (0, 0)),
            pl.BlockSpec((NC, br), lambda i: (0, i)),
        ],
        out_specs=pl.BlockSpec((br, d), lambda i: (i, 0)),
        out_shape=jax.ShapeDtypeStruct((n, d), jnp.float32),
    )(x, W, degp)


def _tcb_body(accp_ref, g_ref, degp_ref, b_ref, o_ref):
    deg = jnp.sum(degp_ref[...], axis=0) + 1.0
    dinv = lax.rsqrt(deg)
    z = (accp_ref[0] + accp_ref[1] + g_ref[...]) * dinv[:, None] + b_ref[...]
    m = jnp.max(z, axis=1, keepdims=True)
    ez = jnp.exp(z - m)
    o_ref[...] = z - m - jnp.log(jnp.sum(ez, axis=1, keepdims=True))


def _tc_finalize(accp, g, degp, b):
    n, d = g.shape
    br = 512
    return pl.pallas_call(
        _tcb_body,
        grid=(pl.cdiv(n, br),),
        in_specs=[
            pl.BlockSpec((NC, br, d), lambda i: (0, i, 0)),
            pl.BlockSpec((br, d), lambda i: (i, 0)),
            pl.BlockSpec((NC, br), lambda i: (0, i)),
            pl.BlockSpec((1, d), lambda i: (0, 0)),
        ],
        out_specs=pl.BlockSpec((br, d), lambda i: (i, 0)),
        out_shape=jax.ShapeDtypeStruct((n, d), jnp.float32),
    )(accp, g, degp, b.reshape(1, d))


def kernel(x, edge_index, W, b):
    n, d_in = x.shape
    d = W.shape[1]
    e = edge_index.shape[1]
    src = edge_index[0]
    dst = edge_index[1]

    chunk, nch, pad = _plan_edges(e)
    if pad:
        ar = jnp.arange(pad, dtype=jnp.int32)
        src = jnp.concatenate([src, ar % n])
        dst = jnp.concatenate([dst, n + (ar % 64)])
        nrows = _round_up(n + 64, 256)
    else:
        nrows = _round_up(n, 256)

    src3 = src.reshape(NW, nch, chunk)
    dst3 = dst.reshape(NW, nch, chunk)

    degp = _sc_degree(dst3, nrows)
    g = _tc_transform(x, W, degp)
    accp = _sc_scatter(g, src3, dst3, nrows)
    out = _tc_finalize(accp, g, degp, b)
    return out
